# baseline (speedup 1.0000x reference)
import functools

import jax
import jax.numpy as jnp
from jax import lax
from jax.experimental import pallas as pl
from jax.experimental.pallas import tpu as pltpu
from jax.experimental.pallas import tpu_sc as plsc

VOCAB = 100000
HIDDEN = 768
BATCH = 4
SEQ = 4096
EPS = 1e-7

_NC = 2
_NS = 16
_NW = _NC * _NS
_B = BATCH * SEQ
_PER_W = _B // _NW
_CHUNK = 16
_NCHUNK = _PER_W // _CHUNK
_NBUF = 4
_LANES = 16
_NVEC = HIDDEN // _LANES


def _rsqrt_scalar(x):
    i = lax.bitcast_convert_type(x, jnp.int32)
    i = jnp.int32(0x5F3759DF) - lax.shift_right_logical(i, 1)
    y = lax.bitcast_convert_type(i, jnp.float32)
    half = x * 0.5
    for _ in range(2):
        y = y * (1.5 - half * y * y)
    return y


def _tree_sum(vals):
    while len(vals) > 1:
        nxt = [a + b for a, b in zip(vals[0::2], vals[1::2])]
        if len(vals) % 2:
            nxt.append(vals[-1])
        vals = nxt
    return vals[0]


def _sc_lookup_ln(ids_flat, word_table):
    mesh = plsc.VectorSubcoreMesh(core_axis_name="c", subcore_axis_name="s")

    @functools.partial(
        pl.kernel,
        mesh=mesh,
        out_type=jax.ShapeDtypeStruct((_B, HIDDEN), jnp.float32),
        scratch_types=(
            [pltpu.VMEM((_CHUNK,), jnp.int32) for _ in range(_NBUF)]
            + [pltpu.VMEM((_CHUNK, HIDDEN), jnp.float32) for _ in range(2 * _NBUF)]
            + [pltpu.SemaphoreType.DMA for _ in range(2 * _NBUF)]
        ),
    )
    def k(ids_hbm, table_hbm, out_hbm, *bufs):
        idx = bufs[0:_NBUF]
        rows = bufs[_NBUF:2 * _NBUF]
        outs = bufs[2 * _NBUF:3 * _NBUF]
        gsem = bufs[3 * _NBUF:4 * _NBUF]
        wsem = bufs[4 * _NBUF:5 * _NBUF]
        wid = lax.axis_index("s") * _NC + lax.axis_index("c")
        base = wid * _PER_W

        def row_norm(rows_v, out_v, r, mv, rstd):
            for j in range(_NVEC):
                x = rows_v[r, pl.ds(j * _LANES, _LANES)]
                out_v[r, pl.ds(j * _LANES, _LANES)] = (x - mv) * rstd

        def ln_chunk(rows_v, out_v):
            def body(r, carry):
                mv_p, rstd_p = carry
                s = jnp.zeros((_LANES,), jnp.float32)
                ss = jnp.zeros((_LANES,), jnp.float32)
                for j in range(_NVEC):
                    x = rows_v[r, pl.ds(j * _LANES, _LANES)]
                    s = s + x
                    ss = ss + x * x
                tot = _tree_sum([s[i] for i in range(_LANES)])
                tss = _tree_sum([ss[i] for i in range(_LANES)])
                mean = tot * (1.0 / HIDDEN)
                var = tss * (1.0 / HIDDEN) - mean * mean
                mv = jnp.full((_LANES,), mean, jnp.float32)
                rstd = jnp.full(
                    (_LANES,), _rsqrt_scalar(var + EPS), jnp.float32
                )
                row_norm(rows_v, out_v, jnp.maximum(r - 1, 0), mv_p, rstd_p)
                return mv, rstd

            zero = jnp.zeros((_LANES,), jnp.float32)
            one = jnp.ones((_LANES,), jnp.float32)
            mv_l, rstd_l = lax.fori_loop(0, _CHUNK, body, (zero, one))
            row_norm(rows_v, out_v, _CHUNK - 1, mv_l, rstd_l)

        def gather_wait(b):
            pltpu.make_async_copy(
                table_hbm.at[idx[b]], rows[b], gsem[b]
            ).wait()

        def wb_wait(b):
            pltpu.make_async_copy(
                outs[b], out_hbm.at[pl.ds(base, _CHUNK)], wsem[b]
            ).wait()

        for b in range(_NBUF - 1):
            pltpu.sync_copy(ids_hbm.at[pl.ds(base + b * _CHUNK, _CHUNK)], idx[b])
            pltpu.async_copy(table_hbm.at[idx[b]], rows[b], gsem[b])

        def group_body(cc, carry):
            for b in range(_NBUF):
                c = _NBUF * cc + b
                gather_wait(b)
                ln_chunk(rows[b], outs[b])
                pltpu.async_copy(
                    outs[b],
                    out_hbm.at[pl.ds(base + c * _CHUNK, _CHUNK)],
                    wsem[b],
                )
                cp = c + _NBUF - 1
                sp = (b + _NBUF - 1) % _NBUF

                @pl.when(jnp.logical_and(cp >= _NBUF, cp < _NCHUNK))
                def _():
                    wb_wait(sp)

                @pl.when(cp < _NCHUNK)
                def _():
                    pltpu.sync_copy(
                        ids_hbm.at[pl.ds(base + cp * _CHUNK, _CHUNK)], idx[sp]
                    )
                    pltpu.async_copy(table_hbm.at[idx[sp]], rows[sp], gsem[sp])

            return carry

        lax.fori_loop(0, _NCHUNK // _NBUF, group_body, 0)

        for b in range(_NBUF):
            wb_wait(b)

    return k(ids_flat, word_table)


def _tc_rel_ln(rel, gamma, beta):
    def body(r_ref, g_ref, b_ref, o_ref):
        x = r_ref[...]
        mean = jnp.mean(x, axis=-1, keepdims=True)
        var = jnp.mean((x - mean) * (x - mean), axis=-1, keepdims=True)
        y = (x - mean) * lax.rsqrt(var + EPS)
        o_ref[...] = y * g_ref[...] + b_ref[...]

    return pl.pallas_call(
        body,
        out_shape=jax.ShapeDtypeStruct(rel.shape, jnp.float32),
    )(rel, gamma, beta)


@jax.jit
def kernel(input_ids, word_table, relative_embedding, rel_ln_gamma, rel_ln_beta):
    ids_flat = input_ids.reshape(-1).astype(jnp.int32)
    word_embedding = _sc_lookup_ln(ids_flat, word_table)
    word_embedding = word_embedding.reshape(BATCH, SEQ, HIDDEN)
    relative_embeddings = _tc_rel_ln(
        relative_embedding, rel_ln_gamma, rel_ln_beta
    )
    return (word_embedding, relative_embeddings)


# speedup vs baseline: 1.1082x; 1.1082x over previous
import functools

import jax
import jax.numpy as jnp
from jax import lax
from jax.experimental import pallas as pl
from jax.experimental.pallas import tpu as pltpu
from jax.experimental.pallas import tpu_sc as plsc

VOCAB = 100000
HIDDEN = 768
BATCH = 4
SEQ = 4096
EPS = 1e-7

_NC = 2
_NS = 16
_NW = _NC * _NS
_B = BATCH * SEQ
_PER_W = _B // _NW
_CHUNK = 32
_NCHUNK = _PER_W // _CHUNK
_NBUF = 4
_LANES = 16
_NVEC = HIDDEN // _LANES


def _rsqrt_scalar(x):
    i = lax.bitcast_convert_type(x, jnp.int32)
    i = jnp.int32(0x5F3759DF) - lax.shift_right_logical(i, 1)
    y = lax.bitcast_convert_type(i, jnp.float32)
    half = x * 0.5
    for _ in range(2):
        y = y * (1.5 - half * y * y)
    return y


def _tree_sum(vals):
    while len(vals) > 1:
        nxt = [a + b for a, b in zip(vals[0::2], vals[1::2])]
        if len(vals) % 2:
            nxt.append(vals[-1])
        vals = nxt
    return vals[0]


def _sc_lookup_ln(ids_flat, word_table):
    mesh = plsc.VectorSubcoreMesh(core_axis_name="c", subcore_axis_name="s")

    @functools.partial(
        pl.kernel,
        mesh=mesh,
        out_type=jax.ShapeDtypeStruct((_B, HIDDEN), jnp.float32),
        scratch_types=(
            [pltpu.VMEM((_CHUNK,), jnp.int32) for _ in range(_NBUF)]
            + [pltpu.VMEM((_CHUNK, HIDDEN), jnp.float32) for _ in range(_NBUF)]
            + [pltpu.SemaphoreType.DMA for _ in range(2 * _NBUF)]
        ),
    )
    def k(ids_hbm, table_hbm, out_hbm, *bufs):
        idx = bufs[0:_NBUF]
        rows = bufs[_NBUF:2 * _NBUF]
        gsem = bufs[2 * _NBUF:3 * _NBUF]
        wsem = bufs[3 * _NBUF:4 * _NBUF]
        wid = lax.axis_index("s") * _NC + lax.axis_index("c")
        base = wid * _PER_W

        def row_norm(rows_v, r, mv, rstd):
            for j in range(_NVEC):
                x = rows_v[r, pl.ds(j * _LANES, _LANES)]
                rows_v[r, pl.ds(j * _LANES, _LANES)] = (x - mv) * rstd

        def ln_chunk(rows_v):
            def body(r, carry):
                mv_p, rstd_p = carry
                s = jnp.zeros((_LANES,), jnp.float32)
                ss = jnp.zeros((_LANES,), jnp.float32)
                for j in range(_NVEC):
                    x = rows_v[r, pl.ds(j * _LANES, _LANES)]
                    s = s + x
                    ss = ss + x * x
                tot = _tree_sum([s[i] for i in range(_LANES)])
                tss = _tree_sum([ss[i] for i in range(_LANES)])
                mean = tot * (1.0 / HIDDEN)
                var = tss * (1.0 / HIDDEN) - mean * mean
                mv = jnp.full((_LANES,), mean, jnp.float32)
                rstd = jnp.full(
                    (_LANES,), _rsqrt_scalar(var + EPS), jnp.float32
                )
                row_norm(rows_v, jnp.maximum(r - 1, 0), mv_p, rstd_p)
                return mv, rstd

            zero = jnp.zeros((_LANES,), jnp.float32)
            one = jnp.ones((_LANES,), jnp.float32)
            mv_l, rstd_l = lax.fori_loop(0, _CHUNK, body, (zero, one))
            row_norm(rows_v, _CHUNK - 1, mv_l, rstd_l)

        def gather_wait(b):
            pltpu.make_async_copy(
                table_hbm.at[idx[b]], rows[b], gsem[b]
            ).wait()

        def wb_wait(b):
            pltpu.make_async_copy(
                rows[b], out_hbm.at[pl.ds(base, _CHUNK)], wsem[b]
            ).wait()

        for b in range(_NBUF - 1):
            pltpu.sync_copy(ids_hbm.at[pl.ds(base + b * _CHUNK, _CHUNK)], idx[b])
            pltpu.async_copy(table_hbm.at[idx[b]], rows[b], gsem[b])

        def group_body(cc, carry):
            for b in range(_NBUF):
                c = _NBUF * cc + b
                gather_wait(b)
                ln_chunk(rows[b])
                pltpu.async_copy(
                    rows[b],
                    out_hbm.at[pl.ds(base + c * _CHUNK, _CHUNK)],
                    wsem[b],
                )
                cp = c + _NBUF - 1
                sp = (b + _NBUF - 1) % _NBUF

                @pl.when(jnp.logical_and(cp >= _NBUF, cp < _NCHUNK))
                def _():
                    wb_wait(sp)

                @pl.when(cp < _NCHUNK)
                def _():
                    pltpu.sync_copy(
                        ids_hbm.at[pl.ds(base + cp * _CHUNK, _CHUNK)], idx[sp]
                    )
                    pltpu.async_copy(table_hbm.at[idx[sp]], rows[sp], gsem[sp])

            return carry

        lax.fori_loop(0, _NCHUNK // _NBUF, group_body, 0)

        for b in range(_NBUF):
            wb_wait(b)

    return k(ids_flat, word_table)


def _tc_rel_ln(rel, gamma, beta):
    def body(r_ref, g_ref, b_ref, o_ref):
        x = r_ref[...]
        mean = jnp.mean(x, axis=-1, keepdims=True)
        var = jnp.mean((x - mean) * (x - mean), axis=-1, keepdims=True)
        y = (x - mean) * lax.rsqrt(var + EPS)
        o_ref[...] = y * g_ref[...] + b_ref[...]

    return pl.pallas_call(
        body,
        out_shape=jax.ShapeDtypeStruct(rel.shape, jnp.float32),
    )(rel, gamma, beta)


@jax.jit
def kernel(input_ids, word_table, relative_embedding, rel_ln_gamma, rel_ln_beta):
    ids_flat = input_ids.reshape(-1).astype(jnp.int32)
    word_embedding = _sc_lookup_ln(ids_flat, word_table)
    word_embedding = word_embedding.reshape(BATCH, SEQ, HIDDEN)
    relative_embeddings = _tc_rel_ln(
        relative_embedding, rel_ln_gamma, rel_ln_beta
    )
    return (word_embedding, relative_embeddings)
